# Initial kernel scaffold; baseline (speedup 1.0000x reference)
import jax
import jax.numpy as jnp
from jax.experimental import pallas as pl


def kernel(xq, xk, xv, wq, bq, wk, bk, wv, bv, wc, bc):
    raise NotImplementedError("write your pallas kernel here")


# speedup vs baseline: 20.1329x; 20.1329x over previous
import functools
import math

import jax
import jax.numpy as jnp
from jax import lax
from jax.experimental import pallas as pl
from jax.experimental.pallas import tpu as pltpu

_VMEM_LIMIT = 56 * 1024 * 1024
_GH = 16


def _mha_kernel(xq_ref, xk_ref, xv_ref,
                wq_ref, bq_ref, wk_ref, wv_ref,
                wc_ref, bc_ref, o_ref,
                xq_bf, xk_bf, xv_bf, c_sc, *, dt):
    j = pl.program_id(1)
    nj = pl.num_programs(1)

    @pl.when(j == 0)
    def _():
        xq_bf[...] = xq_ref[0].astype(jnp.bfloat16)
        xk_bf[...] = xk_ref[0].astype(jnp.bfloat16)
        xv_bf[...] = xv_ref[0].astype(jnp.bfloat16)
        o_ref[0] = jnp.zeros_like(o_ref[0])

    q = (jnp.dot(xq_bf[...], wq_ref[0], preferred_element_type=jnp.float32)
         + bq_ref[0]).astype(jnp.bfloat16)
    k = jnp.dot(xk_bf[...], wk_ref[0],
                preferred_element_type=jnp.float32).astype(jnp.bfloat16)
    v = jnp.dot(xv_bf[...], wv_ref[0],
                preferred_element_type=jnp.float32).astype(jnp.bfloat16)

    for hh in range(_GH):
        sl = slice(hh * dt, (hh + 1) * dt)
        s = lax.dot_general(q[:, sl], k[:, sl], (((1,), (1,)), ((), ())),
                            preferred_element_type=jnp.float32)
        p = jnp.exp(jnp.minimum(s, 80.0))
        l = jnp.sum(p, axis=-1, keepdims=True)
        c = jnp.dot(p.astype(jnp.bfloat16), v[:, sl],
                    preferred_element_type=jnp.float32)
        c_sc[:, sl] = (c / l).astype(jnp.bfloat16)

    o_ref[0] += jnp.dot(c_sc[...], wc_ref[0],
                        preferred_element_type=jnp.float32)

    @pl.when(j == nj - 1)
    def _():
        o_ref[0] += bc_ref[...]


def kernel(xq, xk, xv, wq, bq, wk, bk, wv, bv, wc, bc):
    B, L, D = xq.shape
    H = 16
    dt = D // H
    G = H // _GH
    dg = _GH * dt
    scale = 1.0 / math.sqrt(dt)

    def w_groups(w):
        return w.reshape(D, G, dg).transpose(1, 0, 2).astype(jnp.bfloat16)

    def b_groups(b):
        return b.reshape(1, G, dg).transpose(1, 0, 2)

    wq_p = w_groups(wq * scale)
    wk_p, wv_p = w_groups(wk), w_groups(wv)
    bq_p = b_groups(bq * scale)
    wc_p = wc.reshape(G, dg, D).astype(jnp.bfloat16)
    bc_eff = bc + bv @ wc

    xspec = pl.BlockSpec((1, L, D), lambda b, j: (b, 0, 0))
    wspec = pl.BlockSpec((1, D, dg), lambda b, j: (j, 0, 0))
    bspec = pl.BlockSpec((1, 1, dg), lambda b, j: (j, 0, 0))

    return pl.pallas_call(
        functools.partial(_mha_kernel, dt=dt),
        out_shape=jax.ShapeDtypeStruct((B, L, D), xq.dtype),
        grid=(B, G),
        in_specs=[
            xspec, xspec, xspec,
            wspec, bspec, wspec, wspec,
            pl.BlockSpec((1, dg, D), lambda b, j: (j, 0, 0)),
            pl.BlockSpec((1, D), lambda b, j: (0, 0)),
        ],
        out_specs=pl.BlockSpec((1, L, D), lambda b, j: (b, 0, 0)),
        scratch_shapes=[pltpu.VMEM((L, D), jnp.bfloat16),
                        pltpu.VMEM((L, D), jnp.bfloat16),
                        pltpu.VMEM((L, D), jnp.bfloat16),
                        pltpu.VMEM((L, dg), jnp.bfloat16)],
        compiler_params=pltpu.CompilerParams(
            dimension_semantics=("parallel", "arbitrary"),
            vmem_limit_bytes=_VMEM_LIMIT),
        cost_estimate=pl.CostEstimate(
            flops=2 * B * L * D * (3 * D + D) + 4 * B * H * L * L * dt,
            transcendentals=B * H * L * L,
            bytes_accessed=4 * (4 * B * L * D) + 2 * (4 * D * D)),
    )(xq, xk, xv, wq_p, bq_p, wk_p, wv_p, wc_p, bc_eff)


# speedup vs baseline: 20.3311x; 1.0098x over previous
import functools
import math

import jax
import jax.numpy as jnp
from jax import lax
from jax.experimental import pallas as pl
from jax.experimental.pallas import tpu as pltpu

_VMEM_LIMIT = 56 * 1024 * 1024
_H = 16


def _mha_kernel(xq_ref, xk_ref, xv_ref,
                wq_ref, bq_ref, wk_ref, wv_ref, bv_ref,
                wc_ref, bc_ref, o_ref,
                c_sc, *, dt, qscale):
    xbf = xq_ref[0].astype(jnp.bfloat16)
    q = ((jnp.dot(xbf, wq_ref[...], preferred_element_type=jnp.float32)
          + bq_ref[...]) * qscale).astype(jnp.bfloat16)
    xbf = xk_ref[0].astype(jnp.bfloat16)
    k = jnp.dot(xbf, wk_ref[...],
                preferred_element_type=jnp.float32).astype(jnp.bfloat16)
    xbf = xv_ref[0].astype(jnp.bfloat16)
    v = (jnp.dot(xbf, wv_ref[...], preferred_element_type=jnp.float32)
         + bv_ref[...]).astype(jnp.bfloat16)

    for hh in range(_H):
        sl = slice(hh * dt, (hh + 1) * dt)
        s = lax.dot_general(q[:, sl], k[:, sl], (((1,), (1,)), ((), ())),
                            preferred_element_type=jnp.float32)
        p = jnp.exp2(jnp.minimum(s, 115.0))
        l = jnp.sum(p, axis=-1, keepdims=True)
        c = jnp.dot(p.astype(jnp.bfloat16), v[:, sl],
                    preferred_element_type=jnp.float32)
        c_sc[:, sl] = (c / l).astype(jnp.bfloat16)

    o_ref[0] = (jnp.dot(c_sc[...], wc_ref[...],
                        preferred_element_type=jnp.float32)
                + bc_ref[...])


def kernel(xq, xk, xv, wq, bq, wk, bk, wv, bv, wc, bc):
    B, L, D = xq.shape
    dt = D // _H
    qscale = 1.4426950408889634 / math.sqrt(dt)

    wq_p = wq.astype(jnp.bfloat16)
    wk_p = wk.astype(jnp.bfloat16)
    wv_p = wv.astype(jnp.bfloat16)
    wc_p = wc.astype(jnp.bfloat16)

    xspec = pl.BlockSpec((1, L, D), lambda b: (b, 0, 0))
    wspec = pl.BlockSpec((D, D), lambda b: (0, 0))
    bspec = pl.BlockSpec((1, D), lambda b: (0, 0))

    return pl.pallas_call(
        functools.partial(_mha_kernel, dt=dt, qscale=qscale),
        out_shape=jax.ShapeDtypeStruct((B, L, D), xq.dtype),
        grid=(B,),
        in_specs=[
            xspec, xspec, xspec,
            wspec, bspec, wspec, wspec, bspec,
            wspec, bspec,
        ],
        out_specs=pl.BlockSpec((1, L, D), lambda b: (b, 0, 0)),
        scratch_shapes=[pltpu.VMEM((L, D), jnp.bfloat16)],
        compiler_params=pltpu.CompilerParams(
            dimension_semantics=("parallel",),
            vmem_limit_bytes=_VMEM_LIMIT),
        cost_estimate=pl.CostEstimate(
            flops=2 * B * L * D * (3 * D + D) + 4 * B * _H * L * L * dt,
            transcendentals=B * _H * L * L,
            bytes_accessed=4 * (4 * B * L * D) + 2 * (4 * D * D)),
    )(xq, xk, xv, wq_p, bq, wk_p, wv_p, bv, wc_p, bc)
